# Initial kernel scaffold; baseline (speedup 1.0000x reference)
import jax
import jax.numpy as jnp
from jax.experimental import pallas as pl


def kernel(x_drug, x_target, edge_dd, edge_dt, edge_rev, edge_tt, params):
    raise NotImplementedError("write your pallas kernel here")


# speedup vs baseline: 1.2636x; 1.2636x over previous
import functools
import jax
import jax.numpy as jnp
from jax import lax
from jax.experimental import pallas as pl
from jax.experimental.pallas import tpu as pltpu

N = 50000
D = 128
BM = 512
NPAD = 50176


def _dense_tc_kernel(xd_ref, xt_ref,
                     wsrc_dd, wdst_dd, asrc_dd, adst_dd,
                     wsrc_dt, wdst_dt, asrc_dt, adst_dt,
                     wsrc_rev, wdst_rev, asrc_rev, adst_rev,
                     wsrc_tt, wdst_tt, asrc_tt, adst_tt,
                     h_dd, h_dt, h_rev, h_tt, a8_d, a8_t):
    xd = xd_ref[...]
    xt = xt_ref[...]

    def proj(x, w_ref):
        return jnp.dot(x, w_ref[...], preferred_element_type=jnp.float32)

    def arow(a_ref, h):
        return lax.dot_general(a_ref[...], h, (((1,), (1,)), ((), ())),
                               preferred_element_type=jnp.float32)

    hdd = proj(xd, wsrc_dd)
    hdt = proj(xd, wsrc_dt)
    hrev = proj(xt, wsrc_rev)
    htt = proj(xt, wsrc_tt)
    h_dd[...] = hdd
    h_dt[...] = hdt
    h_rev[...] = hrev
    h_tt[...] = htt

    as_dd = arow(asrc_dd, hdd)
    ad_dd = arow(adst_dd, proj(xd, wdst_dd))
    as_dt = arow(asrc_dt, hdt)
    ad_rev = arow(adst_rev, proj(xd, wdst_rev))
    as_rev = arow(asrc_rev, hrev)
    ad_dt = arow(adst_dt, proj(xt, wdst_dt))
    as_tt = arow(asrc_tt, htt)
    ad_tt = arow(adst_tt, proj(xt, wdst_tt))

    zero = jnp.zeros_like(as_dd)
    a8_d[...] = jnp.concatenate(
        [as_dd, ad_dd, as_dt, ad_rev, zero, zero, zero, zero], axis=0)
    a8_t[...] = jnp.concatenate(
        [as_rev, ad_dt, as_tt, ad_tt, zero, zero, zero, zero], axis=0)


def _dense_layer(xd, xt, p):
    grid = NPAD // BM
    row_spec = pl.BlockSpec((BM, D), lambda i: (i, 0))
    w_spec = pl.BlockSpec((D, D), lambda i: (0, 0))
    a_spec = pl.BlockSpec((1, D), lambda i: (0, 0))
    a8_spec = pl.BlockSpec((8, BM), lambda i: (0, i))

    in_specs = [row_spec, row_spec]
    ops = []
    for c in ("dd", "dt", "rev", "tt"):
        ops += [p[c]["W_src"], p[c]["W_dst"],
                p[c]["a_src"].reshape(1, D), p[c]["a_dst"].reshape(1, D)]
        in_specs += [w_spec, w_spec, a_spec, a_spec]

    out_shapes = [jax.ShapeDtypeStruct((NPAD, D), jnp.float32)] * 4 + \
                 [jax.ShapeDtypeStruct((8, NPAD), jnp.float32)] * 2
    out_specs = [row_spec] * 4 + [a8_spec] * 2

    return pl.pallas_call(
        _dense_tc_kernel,
        grid=(grid,),
        in_specs=in_specs,
        out_specs=out_specs,
        out_shape=out_shapes,
        compiler_params=pltpu.CompilerParams(
            dimension_semantics=("arbitrary",)),
    )(xd, xt, *ops)


def _edge_softmax_scatter(h, a_s, a_d, src, dst, bias_acc):
    e = a_s[src] + a_d[dst]
    e = jnp.where(e > 0, e, 0.2 * e)
    ex = jnp.exp(e)
    denom = jax.ops.segment_sum(ex, dst, num_segments=N)
    alpha = ex / jnp.maximum(denom, 1e-30)[dst]
    msgs = h[src] * alpha[:, None]
    return bias_acc + jax.ops.segment_sum(msgs, dst, num_segments=N)


def kernel(x_drug, x_target, edge_dd, edge_dt, edge_rev, edge_tt, params):
    pad = NPAD - N
    xd = jnp.pad(x_drug, ((0, pad), (0, 0)))
    xt = jnp.pad(x_target, ((0, pad), (0, 0)))

    for p in params:
        h_dd, h_dt, h_rev, h_tt, a8_d, a8_t = _dense_layer(xd, xt, p)

        od = _edge_softmax_scatter(
            h_dd[:N], a8_d[0, :N], a8_d[1, :N], edge_dd[0], edge_dd[1],
            (p["dd"]["b"] + p["rev"]["b"]))
        od = od + _edge_softmax_scatter(
            h_rev[:N], a8_t[0, :N], a8_d[3, :N], edge_rev[0], edge_rev[1], 0.0)
        ot = _edge_softmax_scatter(
            h_dt[:N], a8_d[2, :N], a8_t[1, :N], edge_dt[0], edge_dt[1],
            (p["dt"]["b"] + p["tt"]["b"]))
        ot = ot + _edge_softmax_scatter(
            h_tt[:N], a8_t[2, :N], a8_t[3, :N], edge_tt[0], edge_tt[1], 0.0)

        xd = jnp.pad(jax.nn.relu(od), ((0, pad), (0, 0)))
        xt = jnp.pad(jax.nn.relu(ot), ((0, pad), (0, 0)))

    return (xd[:N], xt[:N])


# speedup vs baseline: 3.8368x; 3.0365x over previous
import functools
import jax
import jax.numpy as jnp
from jax import lax
from jax.experimental import pallas as pl
from jax.experimental.pallas import tpu as pltpu
from jax.experimental.pallas import tpu_sc as plsc

N = 50000
D = 128
E = 150000
BM = 512
NPAD = 50176
EP = 163840
TK = EP // 16
WIN = 1024
NWIN = TK // WIN
R = NPAD // 4
STR = R // 16
CAP = WIN + 16
GW = 128

_mesh = functools.partial(
    plsc.VectorSubcoreMesh, core_axis_name="c", subcore_axis_name="s",
    num_cores=2, num_subcores=16)


def _dense_tc_kernel(xd_ref, xt_ref,
                     wsrc_dd, wdst_dd, asrc_dd, adst_dd,
                     wsrc_dt, wdst_dt, asrc_dt, adst_dt,
                     wsrc_rev, wdst_rev, asrc_rev, adst_rev,
                     wsrc_tt, wdst_tt, asrc_tt, adst_tt,
                     h_dd, h_dt, h_rev, h_tt, a8_d, a8_t):
    xd = xd_ref[...]
    xt = xt_ref[...]

    def proj(x, w_ref):
        return jnp.dot(x, w_ref[...], preferred_element_type=jnp.float32)

    def arow(a_ref, h):
        return lax.dot_general(a_ref[...], h, (((1,), (1,)), ((), ())),
                               preferred_element_type=jnp.float32)

    hdd = proj(xd, wsrc_dd)
    hdt = proj(xd, wsrc_dt)
    hrev = proj(xt, wsrc_rev)
    htt = proj(xt, wsrc_tt)
    h_dd[...] = hdd
    h_dt[...] = hdt
    h_rev[...] = hrev
    h_tt[...] = htt

    as_dd = arow(asrc_dd, hdd)
    ad_dd = arow(adst_dd, proj(xd, wdst_dd))
    as_dt = arow(asrc_dt, hdt)
    ad_rev = arow(adst_rev, proj(xd, wdst_rev))
    as_rev = arow(asrc_rev, hrev)
    ad_dt = arow(adst_dt, proj(xt, wdst_dt))
    as_tt = arow(asrc_tt, htt)
    ad_tt = arow(adst_tt, proj(xt, wdst_tt))

    zero = jnp.zeros_like(as_dd)
    a8_d[...] = jnp.concatenate(
        [as_dd, ad_dd, as_dt, ad_rev, zero, zero, zero, zero], axis=0)
    a8_t[...] = jnp.concatenate(
        [as_rev, ad_dt, as_tt, ad_tt, zero, zero, zero, zero], axis=0)


def _dense_layer(xd, xt, p):
    grid = NPAD // BM
    row_spec = pl.BlockSpec((BM, D), lambda i: (i, 0))
    w_spec = pl.BlockSpec((D, D), lambda i: (0, 0))
    a_spec = pl.BlockSpec((1, D), lambda i: (0, 0))
    a8_spec = pl.BlockSpec((8, BM), lambda i: (0, i))

    in_specs = [row_spec, row_spec]
    ops = []
    for c in ("dd", "dt", "rev", "tt"):
        ops += [p[c]["W_src"], p[c]["W_dst"],
                p[c]["a_src"].reshape(1, D), p[c]["a_dst"].reshape(1, D)]
        in_specs += [w_spec, w_spec, a_spec, a_spec]

    out_shapes = [jax.ShapeDtypeStruct((NPAD, D), jnp.float32)] * 4 + \
                 [jax.ShapeDtypeStruct((8, NPAD), jnp.float32)] * 2
    out_specs = [row_spec] * 4 + [a8_spec] * 2

    return pl.pallas_call(
        _dense_tc_kernel,
        grid=(grid,),
        in_specs=in_specs,
        out_specs=out_specs,
        out_shape=out_shapes,
        compiler_params=pltpu.CompilerParams(
            dimension_semantics=("arbitrary",)),
    )(xd, xt, *ops)


@functools.partial(
    pl.kernel,
    out_type=[jax.ShapeDtypeStruct((EP,), jnp.float32)] * 2,
    mesh=_mesh(),
    compiler_params=pltpu.CompilerParams(needs_layout_passes=False),
    scratch_types=[
        pltpu.VMEM_SHARED((NPAD,), jnp.float32),
        pltpu.VMEM((NPAD,), jnp.float32),
        pltpu.VMEM((NPAD,), jnp.float32),
        pltpu.VMEM((WIN,), jnp.int32),
        pltpu.VMEM((WIN,), jnp.int32),
        pltpu.VMEM((WIN,), jnp.float32),
        pltpu.VMEM((GW,), jnp.int32),
        pltpu.VMEM((GW,), jnp.float32),
        pltpu.VMEM((GW,), jnp.float32),
    ],
)
def _k1_scores(asA, adA, srcA, dstA, asB, adB, srcB, dstB,
               alA, alB, den, as_v, ad_v, srcw, dstw, exw, sidx, sval, dnv):
    c = lax.axis_index("c")
    s = lax.axis_index("s")

    def conv(as_h, ad_h, src_h, dst_h, al_h):
        pltpu.sync_copy(as_h, as_v)
        pltpu.sync_copy(ad_h, ad_v)

        def zv(i, _):
            exw[pl.ds(i * 16, 16)] = jnp.zeros((16,), jnp.float32)
            return 0

        lax.fori_loop(0, WIN // 16, zv, 0)
        base = s * (NPAD // 16)
        for off, sz in ((0, WIN), (WIN, WIN), (2 * WIN, WIN),
                        (3 * WIN, NPAD // 16 - 3 * WIN)):
            pltpu.sync_copy(exw.at[pl.ds(0, sz)],
                            den.at[pl.ds(base + off, sz)])
        plsc.subcore_barrier()

        def win1(w, _):
            b = s * TK + w * WIN
            pltpu.sync_copy(src_h.at[pl.ds(b, WIN)], srcw)
            pltpu.sync_copy(dst_h.at[pl.ds(b, WIN)], dstw)

            def grp(k, _):
                sl = pl.ds(k * 16, 16)
                s16 = srcw[sl]
                d16 = jnp.maximum(dstw[sl], 0)
                e = plsc.load_gather(as_v, [s16]) + plsc.load_gather(ad_v, [d16])
                e = jnp.where(e > 0, e, 0.2 * e)
                exw[sl] = jnp.exp(e)
                return 0

            lax.fori_loop(0, WIN // 16, grp, 0)
            pltpu.sync_copy(exw, al_h.at[pl.ds(b, WIN)])

            def sc(q, _):
                def rc(k, _):
                    a = pl.ds(q * GW + k * 16, 16)
                    bb = pl.ds(k * 16, 16)
                    sidx[bb] = dstw[a]
                    sval[bb] = exw[a]
                    return 0

                lax.fori_loop(0, GW // 16, rc, 0)
                pltpu.sync_copy(
                    sval, den.at[plsc.Indices(sidx, ignored_value=-1)],
                    add=True)
                return 0

            lax.fori_loop(0, WIN // GW, sc, 0)
            return 0

        lax.fori_loop(0, NWIN, win1, 0)
        plsc.subcore_barrier()

        def win2(w, _):
            b = s * TK + w * WIN
            pltpu.sync_copy(dst_h.at[pl.ds(b, WIN)], dstw)
            pltpu.sync_copy(al_h.at[pl.ds(b, WIN)], exw)

            def gq(q, _):
                def rc(k, _):
                    a = pl.ds(q * GW + k * 16, 16)
                    sidx[pl.ds(k * 16, 16)] = jnp.maximum(dstw[a], 0)
                    return 0

                lax.fori_loop(0, GW // 16, rc, 0)
                pltpu.sync_copy(den.at[sidx], dnv)

                def dv(k, _):
                    a = pl.ds(q * GW + k * 16, 16)
                    d16 = dnv[pl.ds(k * 16, 16)]
                    exw[a] = exw[a] / jnp.maximum(d16, 1e-30)
                    return 0

                lax.fori_loop(0, GW // 16, dv, 0)
                return 0

            lax.fori_loop(0, WIN // GW, gq, 0)
            pltpu.sync_copy(exw, al_h.at[pl.ds(b, WIN)])
            return 0

        lax.fori_loop(0, NWIN, win2, 0)

    @pl.when(c == 0)
    def _():
        conv(asA, adA, srcA, dstA, alA)

    @pl.when(c == 1)
    def _():
        conv(asB, adB, srcB, dstB, alB)


@functools.partial(
    pl.kernel,
    out_type=jax.ShapeDtypeStruct((NPAD, D), jnp.float32),
    mesh=_mesh(),
    compiler_params=pltpu.CompilerParams(needs_layout_passes=False),
    scratch_types=[
        pltpu.VMEM_SHARED((R, D), jnp.float32),
        pltpu.VMEM((WIN,), jnp.int32),
        pltpu.VMEM((WIN,), jnp.int32),
        pltpu.VMEM((WIN,), jnp.float32),
        pltpu.VMEM((CAP,), jnp.int32),
        pltpu.VMEM((CAP,), jnp.int32),
        pltpu.VMEM((CAP,), jnp.float32),
        pltpu.VMEM((GW,), jnp.int32),
        pltpu.VMEM((GW,), jnp.int32),
        pltpu.VMEM((GW,), jnp.float32),
        pltpu.VMEM((GW, D), jnp.float32),
        pltpu.VMEM((D,), jnp.float32),
    ],
)
def _k3_aggregate(hA, hB, srcA, dstA, exA, srcB, dstB, exB,
                  bias_h, out_h,
                  ch, srcw, dstw, exw, csrc, cdst, cex,
                  gidx, sidx, sval, rows, biasv):
    c = lax.axis_index("c")
    s = lax.axis_index("s")
    pltpu.sync_copy(bias_h, biasv)

    def ci(i, _):
        csrc[pl.ds(i * 16, 16)] = jnp.zeros((16,), jnp.int32)
        return 0

    lax.fori_loop(0, CAP // 16, ci, 0)

    for cc in range(2):
        chunk = c + 2 * cc
        lo = chunk * R

        def zrow(r, _):
            for k in range(8):
                rows[r, pl.ds(k * 16, 16)] = jnp.zeros((16,), jnp.float32)
            return 0

        lax.fori_loop(0, 112, zrow, 0)

        def zcp(q, _):
            pltpu.sync_copy(rows.at[pl.ds(0, 112)],
                            ch.at[pl.ds(s * STR + q * 112, 112)])
            return 0

        lax.fori_loop(0, STR // 112, zcp, 0)
        plsc.subcore_barrier()

        for (src_h, dst_h, ex_h, h_h) in ((srcA, dstA, exA, hA),
                                          (srcB, dstB, exB, hB)):

            def win(w, _):
                b = s * TK + w * WIN
                pltpu.sync_copy(src_h.at[pl.ds(b, WIN)], srcw)
                pltpu.sync_copy(dst_h.at[pl.ds(b, WIN)], dstw)
                pltpu.sync_copy(ex_h.at[pl.ds(b, WIN)], exw)

                def pf(i, _):
                    cdst[pl.ds(i * 16, 16)] = jnp.full((16,), -1, jnp.int32)
                    return 0

                lax.fori_loop(0, CAP // 16, pf, 0)

                def grp(k, off):
                    sl = pl.ds(k * 16, 16)
                    d16 = dstw[sl]
                    m = (d16 >= lo) & (d16 < lo + R)
                    plsc.store_compressed(cdst.at[pl.ds(off, 16)],
                                          d16 - lo, mask=m)
                    plsc.store_compressed(csrc.at[pl.ds(off, 16)],
                                          srcw[sl], mask=m)
                    plsc.store_compressed(cex.at[pl.ds(off, 16)],
                                          exw[sl], mask=m)
                    return off + jnp.max(
                        plsc.all_reduce_population_count(m))

                off = lax.fori_loop(0, WIN // 16, grp, 0)
                nw = (off + GW - 1) // GW

                def proc(g, _):
                    def rc(k, _):
                        a = pl.ds(g * GW + k * 16, 16)
                        bb = pl.ds(k * 16, 16)
                        gidx[bb] = csrc[a]
                        sidx[bb] = cdst[a]
                        sval[bb] = cex[a]
                        return 0

                    lax.fori_loop(0, GW // 16, rc, 0)
                    pltpu.sync_copy(h_h.at[gidx], rows)

                    def scale(q, _):
                        ev = sval[pl.ds(q * 16, 16)]
                        for j in range(16):
                            r = q * 16 + j
                            iv = jnp.full((16,), ev[j])
                            for k in range(8):
                                sl = pl.ds(k * 16, 16)
                                rows[r, sl] = rows[r, sl] * iv
                        return 0

                    lax.fori_loop(0, GW // 16, scale, 0)
                    pltpu.sync_copy(
                        rows, ch.at[plsc.Indices(sidx, ignored_value=-1)],
                        add=True)
                    return 0

                lax.fori_loop(0, nw, proc, 0)
                return 0

            lax.fori_loop(0, NWIN, win, 0)

        plsc.subcore_barrier()

        def dpiece(p, _):
            row0 = s * STR + p * 112
            pltpu.sync_copy(ch.at[pl.ds(row0, 112)], rows.at[pl.ds(0, 112)])

            def drow(r, _):
                for k in range(8):
                    sl = pl.ds(k * 16, 16)
                    v = rows[r, sl] + biasv[sl]
                    rows[r, sl] = jnp.maximum(v, 0.0)
                return 0

            lax.fori_loop(0, 112, drow, 0)
            pltpu.sync_copy(rows.at[pl.ds(0, 112)],
                            out_h.at[pl.ds(lo + row0, 112)])
            return 0

        lax.fori_loop(0, STR // 112, dpiece, 0)


def _pad_edges(edge):
    src = jnp.concatenate(
        [edge[0], jnp.zeros((EP - E,), jnp.int32)])
    dst = jnp.concatenate(
        [edge[1], jnp.full((EP - E,), -1, jnp.int32)])
    return src, dst


def kernel(x_drug, x_target, edge_dd, edge_dt, edge_rev, edge_tt, params):
    pad = NPAD - N
    xd = jnp.pad(x_drug, ((0, pad), (0, 0)))
    xt = jnp.pad(x_target, ((0, pad), (0, 0)))

    sdd, ddd = _pad_edges(edge_dd)
    sdt, ddt = _pad_edges(edge_dt)
    srv, drv = _pad_edges(edge_rev)
    stt, dtt = _pad_edges(edge_tt)
    keep = []
    for p in params:
        h_dd, h_dt, h_rev, h_tt, a8_d, a8_t = _dense_layer(xd, xt, p)

        ex_dd, ex_rev = _k1_scores(
            a8_d[0], a8_d[1], sdd, ddd,
            a8_t[0], a8_d[3], srv, drv)
        ex_dt, ex_tt = _k1_scores(
            a8_d[2], a8_t[1], sdt, ddt,
            a8_t[2], a8_t[3], stt, dtt)

        xd = _k3_aggregate(
            h_dd, h_rev, sdd, ddd, ex_dd, srv, drv, ex_rev,
            p["dd"]["b"] + p["rev"]["b"])
        xt = _k3_aggregate(
            h_dt, h_tt, sdt, ddt, ex_dt, stt, dtt, ex_tt,
            p["dt"]["b"] + p["tt"]["b"])
        keep += [h_dd, h_dt, h_rev, h_tt, a8_d, a8_t,
                 ex_dd, ex_rev, ex_dt, ex_tt]

    out = lax.optimization_barrier((xd[:N], xt[:N], *keep))
    return (out[0], out[1])


# speedup vs baseline: 6.6005x; 1.7203x over previous
import functools
import jax
import jax.numpy as jnp
from jax import lax
from jax.experimental import pallas as pl
from jax.experimental.pallas import tpu as pltpu
from jax.experimental.pallas import tpu_sc as plsc

N = 50000
D = 128
E = 150000
BM = 512
NPAD = 50176
EP = 163840
TK = EP // 16
WIN = 2048
NWIN = TK // WIN
R = NPAD // 4
STR = R // 16
CAP = WIN + 16
GW = 128

_mesh = functools.partial(
    plsc.VectorSubcoreMesh, core_axis_name="c", subcore_axis_name="s",
    num_cores=2, num_subcores=16)


def _dense_tc_kernel(xd_ref, xt_ref,
                     wsrc_dd, wdst_dd, asrc_dd, adst_dd,
                     wsrc_dt, wdst_dt, asrc_dt, adst_dt,
                     wsrc_rev, wdst_rev, asrc_rev, adst_rev,
                     wsrc_tt, wdst_tt, asrc_tt, adst_tt,
                     h_dd, h_dt, h_rev, h_tt, a8_d, a8_t):
    xd = xd_ref[...]
    xt = xt_ref[...]

    def proj(x, w_ref):
        return jnp.dot(x, w_ref[...], preferred_element_type=jnp.float32)

    def arow(a_ref, h):
        return lax.dot_general(a_ref[...], h, (((1,), (1,)), ((), ())),
                               preferred_element_type=jnp.float32)

    hdd = proj(xd, wsrc_dd)
    hdt = proj(xd, wsrc_dt)
    hrev = proj(xt, wsrc_rev)
    htt = proj(xt, wsrc_tt)
    h_dd[...] = hdd
    h_dt[...] = hdt
    h_rev[...] = hrev
    h_tt[...] = htt

    as_dd = arow(asrc_dd, hdd)
    ad_dd = arow(adst_dd, proj(xd, wdst_dd))
    as_dt = arow(asrc_dt, hdt)
    ad_rev = arow(adst_rev, proj(xd, wdst_rev))
    as_rev = arow(asrc_rev, hrev)
    ad_dt = arow(adst_dt, proj(xt, wdst_dt))
    as_tt = arow(asrc_tt, htt)
    ad_tt = arow(adst_tt, proj(xt, wdst_tt))

    zero = jnp.zeros_like(as_dd)
    a8_d[...] = jnp.concatenate(
        [as_dd, ad_dd, as_dt, ad_rev, zero, zero, zero, zero], axis=0)
    a8_t[...] = jnp.concatenate(
        [as_rev, ad_dt, as_tt, ad_tt, zero, zero, zero, zero], axis=0)


def _dense_layer(xd, xt, p):
    grid = NPAD // BM
    row_spec = pl.BlockSpec((BM, D), lambda i: (i, 0))
    w_spec = pl.BlockSpec((D, D), lambda i: (0, 0))
    a_spec = pl.BlockSpec((1, D), lambda i: (0, 0))
    a8_spec = pl.BlockSpec((8, BM), lambda i: (0, i))

    in_specs = [row_spec, row_spec]
    ops = []
    for c in ("dd", "dt", "rev", "tt"):
        ops += [p[c]["W_src"], p[c]["W_dst"],
                p[c]["a_src"].reshape(1, D), p[c]["a_dst"].reshape(1, D)]
        in_specs += [w_spec, w_spec, a_spec, a_spec]

    out_shapes = [jax.ShapeDtypeStruct((NPAD, D), jnp.float32)] * 4 + \
                 [jax.ShapeDtypeStruct((8, NPAD), jnp.float32)] * 2
    out_specs = [row_spec] * 4 + [a8_spec] * 2

    return pl.pallas_call(
        _dense_tc_kernel,
        grid=(grid,),
        in_specs=in_specs,
        out_specs=out_specs,
        out_shape=out_shapes,
        compiler_params=pltpu.CompilerParams(
            dimension_semantics=("arbitrary",)),
    )(xd, xt, *ops)


@functools.partial(
    pl.kernel,
    out_type=[jax.ShapeDtypeStruct((EP,), jnp.float32)] * 2,
    mesh=_mesh(),
    compiler_params=pltpu.CompilerParams(needs_layout_passes=False),
    scratch_types=[
        pltpu.VMEM_SHARED((NPAD,), jnp.float32),
        pltpu.VMEM((NPAD,), jnp.float32),
        pltpu.VMEM((NPAD,), jnp.float32),
        pltpu.VMEM((WIN,), jnp.int32),
        pltpu.VMEM((WIN,), jnp.int32),
        pltpu.VMEM((WIN,), jnp.float32),
        pltpu.VMEM((GW,), jnp.int32),
        pltpu.VMEM((GW,), jnp.float32),
        pltpu.VMEM((GW,), jnp.float32),
    ],
)
def _k1_scores(asA, adA, srcA, dstA, asB, adB, srcB, dstB,
               alA, alB, den, as_v, ad_v, srcw, dstw, exw, sidx, sval, dnv):
    c = lax.axis_index("c")
    s = lax.axis_index("s")

    def conv(as_h, ad_h, src_h, dst_h, al_h):
        pltpu.sync_copy(as_h, as_v)
        pltpu.sync_copy(ad_h, ad_v)

        def zv(i, _):
            exw[pl.ds(i * 16, 16)] = jnp.zeros((16,), jnp.float32)
            return 0

        lax.fori_loop(0, WIN // 16, zv, 0)
        base = s * (NPAD // 16)
        off = 0
        while off < NPAD // 16:
            sz = min(WIN, NPAD // 16 - off)
            pltpu.sync_copy(exw.at[pl.ds(0, sz)],
                            den.at[pl.ds(base + off, sz)])
            off += sz
        plsc.subcore_barrier()

        def win1(w, _):
            b = s * TK + w * WIN
            pltpu.sync_copy(src_h.at[pl.ds(b, WIN)], srcw)
            pltpu.sync_copy(dst_h.at[pl.ds(b, WIN)], dstw)

            def grp(k, _):
                sl = pl.ds(k * 16, 16)
                s16 = srcw[sl]
                d16 = jnp.maximum(dstw[sl], 0)
                e = plsc.load_gather(as_v, [s16]) + plsc.load_gather(ad_v, [d16])
                e = jnp.where(e > 0, e, 0.2 * e)
                exw[sl] = jnp.exp(e)
                return 0

            lax.fori_loop(0, WIN // 16, grp, 0)
            pltpu.sync_copy(exw, al_h.at[pl.ds(b, WIN)])

            def sc(q, _):
                def rc(k, _):
                    a = pl.ds(q * GW + k * 16, 16)
                    bb = pl.ds(k * 16, 16)
                    sidx[bb] = dstw[a]
                    sval[bb] = exw[a]
                    return 0

                lax.fori_loop(0, GW // 16, rc, 0)
                pltpu.sync_copy(
                    sval, den.at[plsc.Indices(sidx, ignored_value=-1)],
                    add=True)
                return 0

            lax.fori_loop(0, WIN // GW, sc, 0)
            return 0

        lax.fori_loop(0, NWIN, win1, 0)
        plsc.subcore_barrier()

        def win2(w, _):
            b = s * TK + w * WIN
            pltpu.sync_copy(dst_h.at[pl.ds(b, WIN)], dstw)
            pltpu.sync_copy(al_h.at[pl.ds(b, WIN)], exw)

            def gq(q, _):
                def rc(k, _):
                    a = pl.ds(q * GW + k * 16, 16)
                    sidx[pl.ds(k * 16, 16)] = jnp.maximum(dstw[a], 0)
                    return 0

                lax.fori_loop(0, GW // 16, rc, 0)
                pltpu.sync_copy(den.at[sidx], dnv)

                def dv(k, _):
                    a = pl.ds(q * GW + k * 16, 16)
                    d16 = dnv[pl.ds(k * 16, 16)]
                    exw[a] = exw[a] / jnp.maximum(d16, 1e-30)
                    return 0

                lax.fori_loop(0, GW // 16, dv, 0)
                return 0

            lax.fori_loop(0, WIN // GW, gq, 0)
            pltpu.sync_copy(exw, al_h.at[pl.ds(b, WIN)])
            return 0

        lax.fori_loop(0, NWIN, win2, 0)

    @pl.when(c == 0)
    def _():
        conv(asA, adA, srcA, dstA, alA)

    @pl.when(c == 1)
    def _():
        conv(asB, adB, srcB, dstB, alB)


@functools.partial(
    pl.kernel,
    out_type=jax.ShapeDtypeStruct((NPAD, D), jnp.float32),
    mesh=_mesh(),
    compiler_params=pltpu.CompilerParams(needs_layout_passes=False),
    scratch_types=[
        pltpu.VMEM_SHARED((R, D), jnp.float32),
        pltpu.VMEM((WIN,), jnp.int32),
        pltpu.VMEM((WIN,), jnp.int32),
        pltpu.VMEM((WIN,), jnp.float32),
        pltpu.VMEM((CAP,), jnp.int32),
        pltpu.VMEM((CAP,), jnp.int32),
        pltpu.VMEM((CAP,), jnp.float32),
        pltpu.VMEM((GW,), jnp.int32),
        pltpu.VMEM((GW,), jnp.int32),
        pltpu.VMEM((GW,), jnp.float32),
        pltpu.VMEM((GW, D), jnp.float32),
        pltpu.VMEM((D,), jnp.float32),
    ],
)
def _k3_aggregate(hA, hB, srcA, dstA, exA, srcB, dstB, exB,
                  bias_h, out_h,
                  ch, srcw, dstw, exw, csrc, cdst, cex,
                  gidx, sidx, sval, rows, biasv):
    c = lax.axis_index("c")
    s = lax.axis_index("s")
    pltpu.sync_copy(bias_h, biasv)

    def ci(i, _):
        csrc[pl.ds(i * 16, 16)] = jnp.zeros((16,), jnp.int32)
        return 0

    lax.fori_loop(0, CAP // 16, ci, 0)

    for cc in range(2):
        chunk = c + 2 * cc
        lo = chunk * R

        def zrow(r, _):
            for k in range(8):
                rows[r, pl.ds(k * 16, 16)] = jnp.zeros((16,), jnp.float32)
            return 0

        lax.fori_loop(0, 112, zrow, 0)

        def zcp(q, _):
            pltpu.sync_copy(rows.at[pl.ds(0, 112)],
                            ch.at[pl.ds(s * STR + q * 112, 112)])
            return 0

        lax.fori_loop(0, STR // 112, zcp, 0)
        plsc.subcore_barrier()

        for (src_h, dst_h, ex_h, h_h) in ((srcA, dstA, exA, hA),
                                          (srcB, dstB, exB, hB)):

            def win(w, _):
                b = s * TK + w * WIN
                pltpu.sync_copy(src_h.at[pl.ds(b, WIN)], srcw)
                pltpu.sync_copy(dst_h.at[pl.ds(b, WIN)], dstw)
                pltpu.sync_copy(ex_h.at[pl.ds(b, WIN)], exw)

                def pf(i, _):
                    cdst[pl.ds(i * 16, 16)] = jnp.full((16,), -1, jnp.int32)
                    return 0

                lax.fori_loop(0, CAP // 16, pf, 0)

                def grp(k, off):
                    sl = pl.ds(k * 16, 16)
                    d16 = dstw[sl]
                    m = (d16 >= lo) & (d16 < lo + R)
                    plsc.store_compressed(cdst.at[pl.ds(off, 16)],
                                          d16 - lo, mask=m)
                    plsc.store_compressed(csrc.at[pl.ds(off, 16)],
                                          srcw[sl], mask=m)
                    plsc.store_compressed(cex.at[pl.ds(off, 16)],
                                          exw[sl], mask=m)
                    return off + jnp.max(
                        plsc.all_reduce_population_count(m))

                off = lax.fori_loop(0, WIN // 16, grp, 0)
                nw = (off + GW - 1) // GW

                def proc(g, _):
                    def rc(k, _):
                        a = pl.ds(g * GW + k * 16, 16)
                        bb = pl.ds(k * 16, 16)
                        gidx[bb] = csrc[a]
                        sidx[bb] = cdst[a]
                        sval[bb] = cex[a]
                        return 0

                    lax.fori_loop(0, GW // 16, rc, 0)
                    pltpu.sync_copy(h_h.at[gidx], rows)

                    def scale(q, _):
                        ev = sval[pl.ds(q * 16, 16)]
                        for j in range(16):
                            r = q * 16 + j
                            iv = jnp.full((16,), ev[j])
                            for k in range(8):
                                sl = pl.ds(k * 16, 16)
                                rows[r, sl] = rows[r, sl] * iv
                        return 0

                    lax.fori_loop(0, GW // 16, scale, 0)
                    pltpu.sync_copy(
                        rows, ch.at[plsc.Indices(sidx, ignored_value=-1)],
                        add=True)
                    return 0

                lax.fori_loop(0, nw, proc, 0)
                return 0

            lax.fori_loop(0, NWIN, win, 0)

        plsc.subcore_barrier()

        def dpiece(p, _):
            row0 = s * STR + p * 112
            pltpu.sync_copy(ch.at[pl.ds(row0, 112)], rows.at[pl.ds(0, 112)])

            def drow(r, _):
                for k in range(8):
                    sl = pl.ds(k * 16, 16)
                    v = rows[r, sl] + biasv[sl]
                    rows[r, sl] = jnp.maximum(v, 0.0)
                return 0

            lax.fori_loop(0, 112, drow, 0)
            pltpu.sync_copy(rows.at[pl.ds(0, 112)],
                            out_h.at[pl.ds(lo + row0, 112)])
            return 0

        lax.fori_loop(0, STR // 112, dpiece, 0)


def _pad_edges(edge):
    src = jnp.concatenate(
        [edge[0], jnp.zeros((EP - E,), jnp.int32)])
    dst = jnp.concatenate(
        [edge[1], jnp.full((EP - E,), -1, jnp.int32)])
    return src, dst


def kernel(x_drug, x_target, edge_dd, edge_dt, edge_rev, edge_tt, params):
    pad = NPAD - N
    xd = jnp.pad(x_drug, ((0, pad), (0, 0)))
    xt = jnp.pad(x_target, ((0, pad), (0, 0)))

    sdd, ddd = _pad_edges(edge_dd)
    sdt, ddt = _pad_edges(edge_dt)
    srv, drv = _pad_edges(edge_rev)
    stt, dtt = _pad_edges(edge_tt)
    keep = []
    for p in params:
        h_dd, h_dt, h_rev, h_tt, a8_d, a8_t = _dense_layer(xd, xt, p)

        ex_dd, ex_rev = _k1_scores(
            a8_d[0], a8_d[1], sdd, ddd,
            a8_t[0], a8_d[3], srv, drv)
        ex_dt, ex_tt = _k1_scores(
            a8_d[2], a8_t[1], sdt, ddt,
            a8_t[2], a8_t[3], stt, dtt)

        xd = _k3_aggregate(
            h_dd, h_rev, sdd, ddd, ex_dd, srv, drv, ex_rev,
            p["dd"]["b"] + p["rev"]["b"])
        xt = _k3_aggregate(
            h_dt, h_tt, sdt, ddt, ex_dt, stt, dtt, ex_tt,
            p["dt"]["b"] + p["tt"]["b"])
        keep += [h_dd, h_dt, h_rev, h_tt, a8_d, a8_t,
                 ex_dd, ex_rev, ex_dt, ex_tt]

    out = lax.optimization_barrier((xd[:N], xt[:N], *keep))
    return (out[0], out[1])
